# baseline (speedup 1.0000x reference)
import functools
import numpy as np
import jax
import jax.numpy as jnp
from jax import lax
from jax.experimental import pallas as pl
from jax.experimental.pallas import tpu as pltpu
from jax.experimental.pallas import tpu_sc as plsc

N = 10000
E = 320000
D = 128
HID = 128
H = 8
C = HID // H
OUT = 128

NC, NS = 2, 16
NW = NC * NS
K = 96
CPW = 106
E2 = K * NW * CPW
NACC = 10240
ZR = 80

RB = 1000


def _pre_body(h_ref, lW_ref, lb_ref, Ws_ref, Wd_ref, bds_ref, bdd_ref,
              xl_ref, xs_ref, ab_ref):
    hb = h_ref[...]
    xl_ref[...] = jnp.dot(hb, lW_ref[...], preferred_element_type=jnp.float32) + lb_ref[...]
    xs = jnp.dot(hb, Ws_ref[...], preferred_element_type=jnp.float32)
    xs_ref[...] = xs
    xd = jnp.dot(hb, Wd_ref[...], preferred_element_type=jnp.float32)
    ab_ref[...] = (jnp.dot(xs, bds_ref[...], preferred_element_type=jnp.float32)
                   + jnp.dot(xd, bdd_ref[...], preferred_element_type=jnp.float32))


def _pre_layer(h, lW, lb, Ws, Wd, bds, bdd):
    grid = (N // RB,)
    full = lambda shape: pl.BlockSpec(shape, lambda i: (0, 0))
    return pl.pallas_call(
        _pre_body,
        grid=grid,
        in_specs=[
            pl.BlockSpec((RB, D), lambda i: (i, 0)),
            full((D, HID)), full((1, HID)), full((D, HID)), full((D, HID)),
            full((HID, C)), full((HID, C)),
        ],
        out_specs=[
            pl.BlockSpec((RB, HID), lambda i: (i, 0)),
            pl.BlockSpec((RB, HID), lambda i: (i, 0)),
            pl.BlockSpec((RB, C), lambda i: (i, 0)),
        ],
        out_shape=[
            jax.ShapeDtypeStruct((N, HID), jnp.float32),
            jax.ShapeDtypeStruct((N, HID), jnp.float32),
            jax.ShapeDtypeStruct((NACC, C), jnp.float32),
        ],
    )(h, lW, lb, Ws, Wd, bds, bdd)


def _sc_edge_body(xs_hbm, ab_hbm, sd_hbm,
                  outS, outD,
                  accS, accD,
                  sd0, rows0, as0, ad0, gsem0, ssem0,
                  sd1, rows1, as1, ad1, gsem1, ssem1):
    c = lax.axis_index("c")
    s = lax.axis_index("s")
    w = c * NS + s
    bufs = ((sd0, rows0, as0, ad0, gsem0, ssem0),
            (sd1, rows1, as1, ad1, gsem1, ssem1))

    def fire(j, b):
        sd_v, rows_v, asv, adv, gsem, _ = b
        pltpu.sync_copy(sd_hbm.at[j * NW + w], sd_v)
        pltpu.async_copy(xs_hbm.at[sd_v.at[0]], rows_v, gsem)
        pltpu.async_copy(ab_hbm.at[sd_v.at[0]], asv, gsem)
        pltpu.async_copy(ab_hbm.at[sd_v.at[1]], adv, gsem)

    def waitg(b):
        sd_v, rows_v, asv, adv, gsem, _ = b
        pltpu.make_async_copy(xs_hbm.at[sd_v.at[0]], rows_v, gsem).wait()
        pltpu.make_async_copy(ab_hbm.at[sd_v.at[0]], asv, gsem).wait()
        pltpu.make_async_copy(ab_hbm.at[sd_v.at[1]], adv, gsem).wait()

    perm = ((lax.iota(jnp.int32, 16) + 8) % 16).reshape(16, 1)
    gdn = lax.GatherDimensionNumbers(
        offset_dims=(), collapsed_slice_dims=(0,), start_index_map=(0,))

    def compute(b):
        sd_v, rows_v, asv, adv, gsem, ssem = b

        @plsc.parallel_loop(0, K, unroll=8)
        def _edge(e):
            adn = lax.gather(adv[e, :], perm, gdn, (1,),
                             mode=lax.GatherScatterMode.PROMISE_IN_BOUNDS)
            a = asv[e, :] + adn
            a = jnp.where(a >= 0, a, 0.2 * a)
            ex = jnp.exp(a)
            adv[e, :] = ex
            for h in range(H):
                rv = rows_v[e, pl.ds(h * 16, 16)]
                rows_v[e, pl.ds(h * 16, 16)] = rv * ex[h]

        pltpu.async_copy(rows_v, accS.at[sd_v.at[1]], ssem, add=True)
        pltpu.async_copy(adv, accD.at[sd_v.at[1]], ssem, add=True)

    def waitsc(b):
        sd_v, rows_v, asv, adv, gsem, ssem = b
        pltpu.make_async_copy(rows_v, accS.at[sd_v.at[1]], ssem).wait()
        pltpu.make_async_copy(adv, accD.at[sd_v.at[1]], ssem).wait()

    fire(0, bufs[0])

    zero16 = jnp.zeros((16,), jnp.float32)

    def _zs(r, _):
        for j in range(HID // 16):
            rows1[r, pl.ds(j * 16, 16)] = zero16
        as1[r, :] = zero16
        return _

    lax.fori_loop(0, ZR, _zs, None)
    r0 = s * (8 * ZR)
    for kk in range(NACC // NS // ZR):
        r = r0 + kk * ZR
        pltpu.sync_copy(rows1.at[pl.ds(0, ZR)], accS.at[pl.ds(r, ZR)])
        pltpu.sync_copy(as1.at[pl.ds(0, ZR)], accD.at[pl.ds(r, ZR)])
    plsc.subcore_barrier()

    def _pair(p, _):
        fire(2 * p + 1, bufs[1])
        waitg(bufs[0])
        compute(bufs[0])
        waitg(bufs[1])
        compute(bufs[1])
        waitsc(bufs[0])

        @pl.when(p < CPW // 2 - 1)
        def _():
            fire(2 * p + 2, bufs[0])

        waitsc(bufs[1])
        return _

    lax.fori_loop(0, CPW // 2, _pair, None)
    plsc.subcore_barrier()

    for kk in range(NACC // NS // ZR):
        r = r0 + kk * ZR
        pltpu.sync_copy(accS.at[pl.ds(r, ZR)], rows0.at[pl.ds(0, ZR)])
        pltpu.sync_copy(rows0.at[pl.ds(0, ZR)], outS.at[c, pl.ds(r, ZR)])
        pltpu.sync_copy(accD.at[pl.ds(r, ZR)], as0.at[pl.ds(0, ZR)])
        pltpu.sync_copy(as0.at[pl.ds(0, ZR)], outD.at[c, pl.ds(r, ZR)])


NCH2 = E2 // K


def _sc_edge(xs, ab16, sd):
    mesh = plsc.VectorSubcoreMesh(core_axis_name="c", subcore_axis_name="s",
                                  num_cores=NC, num_subcores=NS)
    buf = [
        pltpu.VMEM((2, K), jnp.int32),
        pltpu.VMEM((K, HID), jnp.float32),
        pltpu.VMEM((K, C), jnp.float32),
        pltpu.VMEM((K, C), jnp.float32),
        pltpu.SemaphoreType.DMA,
        pltpu.SemaphoreType.DMA,
    ]
    return pl.kernel(
        _sc_edge_body,
        compiler_params=pltpu.CompilerParams(use_tc_tiling_on_sc=False),
        out_type=[
            jax.ShapeDtypeStruct((NC, NACC, HID), jnp.float32),
            jax.ShapeDtypeStruct((NC, NACC, C), jnp.float32),
        ],
        mesh=mesh,
        scratch_types=[
            pltpu.VMEM_SHARED((NACC, HID), jnp.float32),
            pltpu.VMEM_SHARED((NACC, C), jnp.float32),
            *buf, *buf,
        ],
    )(xs, ab16, sd)


def _postpre_body(accS_ref, accD_ref, xl_ref, cb_ref, exp_ref,
                  lW_ref, lb_ref, Ws_ref, Wd_ref, bds_ref, bdd_ref,
                  xl2_ref, xs_ref, ab_ref):
    S = accS_ref[0] + accS_ref[1]
    den = accD_ref[0] + accD_ref[1]
    d128 = jnp.dot(den, exp_ref[...], preferred_element_type=jnp.float32) + 1e-16
    hn = xl_ref[...] + S / d128 + cb_ref[...]
    hb = jnp.where(hn > 0, hn, jnp.exp(hn) - 1.0)
    xl2_ref[...] = jnp.dot(hb, lW_ref[...], preferred_element_type=jnp.float32) + lb_ref[...]
    xs = jnp.dot(hb, Ws_ref[...], preferred_element_type=jnp.float32)
    xs_ref[...] = xs
    xd = jnp.dot(hb, Wd_ref[...], preferred_element_type=jnp.float32)
    ab_ref[...] = (jnp.dot(xs, bds_ref[...], preferred_element_type=jnp.float32)
                   + jnp.dot(xd, bdd_ref[...], preferred_element_type=jnp.float32))


def _postpre_layer(accS, accD, xl, cb, expand, lW, lb, Ws, Wd, bds, bdd):
    grid = (N // RB,)
    full = lambda shape: pl.BlockSpec(shape, lambda i: tuple(0 for _ in shape))
    return pl.pallas_call(
        _postpre_body,
        grid=grid,
        in_specs=[
            pl.BlockSpec((NC, RB, HID), lambda i: (0, i, 0)),
            pl.BlockSpec((NC, RB, C), lambda i: (0, i, 0)),
            pl.BlockSpec((RB, HID), lambda i: (i, 0)),
            full((1, HID)), full((C, HID)),
            full((HID, HID)), full((1, HID)), full((HID, HID)), full((HID, HID)),
            full((HID, C)), full((HID, C)),
        ],
        out_specs=[
            pl.BlockSpec((RB, HID), lambda i: (i, 0)),
            pl.BlockSpec((RB, HID), lambda i: (i, 0)),
            pl.BlockSpec((RB, C), lambda i: (i, 0)),
        ],
        out_shape=[
            jax.ShapeDtypeStruct((N, HID), jnp.float32),
            jax.ShapeDtypeStruct((N, HID), jnp.float32),
            jax.ShapeDtypeStruct((NACC, C), jnp.float32),
        ],
    )(accS, accD, xl, cb, expand, lW, lb, Ws, Wd, bds, bdd)


def _final_body(accS_ref, accD_ref, xl_ref, cb_ref, exp_ref, fW_ref, fb_ref, o_ref):
    S = accS_ref[0] + accS_ref[1]
    den = accD_ref[0] + accD_ref[1]
    d128 = jnp.dot(den, exp_ref[...], preferred_element_type=jnp.float32) + 1e-16
    hn = xl_ref[...] + S / d128 + cb_ref[...]
    hn = jnp.where(hn > 0, hn, jnp.exp(hn) - 1.0)
    o_ref[...] = jnp.dot(hn, fW_ref[...], preferred_element_type=jnp.float32) + fb_ref[...]


def _post_layer(accS, accD, xl, cb, expand, fW, fb):
    grid = (N // RB,)
    full = lambda shape: pl.BlockSpec(shape, lambda i: tuple(0 for _ in shape))
    in_specs = [
        pl.BlockSpec((NC, RB, HID), lambda i: (0, i, 0)),
        pl.BlockSpec((NC, RB, C), lambda i: (0, i, 0)),
        pl.BlockSpec((RB, HID), lambda i: (i, 0)),
        full((1, HID)), full((C, HID)),
    ]
    args = [accS, accD, xl, cb, expand, fW, fb]
    body, out_d = _final_body, OUT
    in_specs += [full((HID, OUT)), full((1, OUT))]
    return pl.pallas_call(
        body,
        grid=grid,
        in_specs=in_specs,
        out_specs=pl.BlockSpec((RB, out_d), lambda i: (i, 0)),
        out_shape=jax.ShapeDtypeStruct((N, out_d), jnp.float32),
    )(*args)


def _blockdiag(att, off):
    bd = jnp.zeros((HID, C), jnp.float32)
    rows = jnp.arange(HID)
    return bd.at[rows, off + rows // C].set(att.reshape(HID))


_EXPAND_NP = np.zeros((C, HID), np.float32)
_EXPAND_NP[np.arange(HID) // C, np.arange(HID)] = 1.0


def kernel(x, edge_index, lW0, lb0, Ws0, Wd0, atts0, attd0, cb0,
           lW1, lb1, Ws1, Wd1, atts1, attd1, cb1, fW, fb):
    srcp = jnp.concatenate([edge_index[0], jnp.zeros((E2 - E,), jnp.int32)])
    dstp = jnp.concatenate([edge_index[1], jnp.full((E2 - E,), N, jnp.int32)])
    sd = jnp.stack([srcp.reshape(NCH2, K), dstp.reshape(NCH2, K)], axis=1)
    expand = jnp.asarray(_EXPAND_NP)

    xl0, xs0, ab0 = _pre_layer(
        x, lW0, lb0.reshape(1, HID), Ws0, Wd0,
        _blockdiag(atts0, 0), _blockdiag(attd0, 8))
    accS0, accD0 = _sc_edge(xs0, ab0, sd)
    xl1, xs1, ab1 = _postpre_layer(
        accS0, accD0, xl0, cb0.reshape(1, HID), expand,
        lW1, lb1.reshape(1, HID), Ws1, Wd1,
        _blockdiag(atts1, 0), _blockdiag(attd1, 8))
    accS1, accD1 = _sc_edge(xs1, ab1, sd)
    return _post_layer(accS1, accD1, xl1, cb1.reshape(1, HID), expand,
                       fW, fb.reshape(1, OUT))


# speedup vs baseline: 1.1242x; 1.1242x over previous
import functools
import numpy as np
import jax
import jax.numpy as jnp
from jax import lax
from jax.experimental import pallas as pl
from jax.experimental.pallas import tpu as pltpu
from jax.experimental.pallas import tpu_sc as plsc

N = 10000
E = 320000
D = 128
HID = 128
H = 8
C = HID // H
OUT = 128

NC, NS = 2, 16
NW = NC * NS
K = 96
CPW = 106
E2 = K * NW * CPW
NACC = 10240
ZR = 80

RB = 1000


def _pre_body(h_ref, lW_ref, lb_ref, Ws_ref, Wd_ref, bds_ref, bdd_ref,
              xl_ref, xs_ref, ab_ref):
    hb = h_ref[...]
    xl_ref[...] = jnp.dot(hb, lW_ref[...], preferred_element_type=jnp.float32) + lb_ref[...]
    xs = jnp.dot(hb, Ws_ref[...], preferred_element_type=jnp.float32)
    xs_ref[...] = xs
    xd = jnp.dot(hb, Wd_ref[...], preferred_element_type=jnp.float32)
    ab_ref[...] = (jnp.dot(xs, bds_ref[...], preferred_element_type=jnp.float32)
                   + jnp.dot(xd, bdd_ref[...], preferred_element_type=jnp.float32))


def _pre_layer(h, lW, lb, Ws, Wd, bds, bdd):
    grid = (N // RB,)
    full = lambda shape: pl.BlockSpec(shape, lambda i: (0, 0))
    return pl.pallas_call(
        _pre_body,
        grid=grid,
        in_specs=[
            pl.BlockSpec((RB, D), lambda i: (i, 0)),
            full((D, HID)), full((1, HID)), full((D, HID)), full((D, HID)),
            full((HID, C)), full((HID, C)),
        ],
        out_specs=[
            pl.BlockSpec((RB, HID), lambda i: (i, 0)),
            pl.BlockSpec((RB, HID), lambda i: (i, 0)),
            pl.BlockSpec((RB, C), lambda i: (i, 0)),
        ],
        out_shape=[
            jax.ShapeDtypeStruct((N, HID), jnp.float32),
            jax.ShapeDtypeStruct((N, HID), jnp.float32),
            jax.ShapeDtypeStruct((NACC, C), jnp.float32),
        ],
    )(h, lW, lb, Ws, Wd, bds, bdd)


def _sc_edge_body(xs_hbm, ab_hbm, sd_hbm,
                  outS, outD,
                  accS, accD,
                  sd0, rows0, as0, ad0, scd0, gsem0, ssem0, isem0,
                  sd1, rows1, as1, ad1, scd1, gsem1, ssem1, isem1):
    c = lax.axis_index("c")
    s = lax.axis_index("s")
    w = c * NS + s
    bufs = ((sd0, rows0, as0, ad0, scd0, gsem0, ssem0, isem0),
            (sd1, rows1, as1, ad1, scd1, gsem1, ssem1, isem1))

    def pfxi(j, b):
        sd_v, isem = b[0], b[7]
        pltpu.async_copy(sd_hbm.at[j * NW + w], sd_v, isem)

    def waiti(j, b):
        sd_v, isem = b[0], b[7]
        pltpu.make_async_copy(sd_hbm.at[j * NW + w], sd_v, isem).wait()

    def fire(j, b):
        sd_v, rows_v, asv, adv, gsem = b[0], b[1], b[2], b[3], b[5]
        pltpu.async_copy(xs_hbm.at[sd_v.at[0]], rows_v, gsem)
        pltpu.async_copy(ab_hbm.at[sd_v.at[0]], asv, gsem)
        pltpu.async_copy(ab_hbm.at[sd_v.at[1]], adv, gsem)

    def waitg(b):
        sd_v, rows_v, asv, adv, gsem = b[0], b[1], b[2], b[3], b[5]
        pltpu.make_async_copy(xs_hbm.at[sd_v.at[0]], rows_v, gsem).wait()
        pltpu.make_async_copy(ab_hbm.at[sd_v.at[0]], asv, gsem).wait()
        pltpu.make_async_copy(ab_hbm.at[sd_v.at[1]], adv, gsem).wait()

    perm = ((lax.iota(jnp.int32, 16) + 8) % 16).reshape(16, 1)
    gdn = lax.GatherDimensionNumbers(
        offset_dims=(), collapsed_slice_dims=(0,), start_index_map=(0,))

    def compute(b):
        sd_v, rows_v, asv, adv, scd, gsem, ssem = b[:7]

        @plsc.parallel_loop(0, K // 16, unroll=2)
        def _cpy(jj):
            scd[pl.ds(jj * 16, 16)] = sd_v[1, pl.ds(jj * 16, 16)]

        @plsc.parallel_loop(0, K, unroll=4)
        def _edge(e):
            adn = lax.gather(adv[e, :], perm, gdn, (1,),
                             mode=lax.GatherScatterMode.PROMISE_IN_BOUNDS)
            a = asv[e, :] + adn
            a = jnp.where(a >= 0, a, 0.2 * a)
            ex = jnp.exp(a)
            adv[e, :] = ex
            for h in range(H):
                rv = rows_v[e, pl.ds(h * 16, 16)]
                rows_v[e, pl.ds(h * 16, 16)] = rv * ex[h]

        pltpu.async_copy(rows_v, accS.at[scd], ssem, add=True)
        pltpu.async_copy(adv, accD.at[scd], ssem, add=True)

    def waitsc(b):
        sd_v, rows_v, asv, adv, scd, gsem, ssem = b[:7]
        pltpu.make_async_copy(rows_v, accS.at[scd], ssem).wait()
        pltpu.make_async_copy(adv, accD.at[scd], ssem).wait()

    pfxi(0, bufs[0])
    pfxi(1, bufs[1])
    waiti(0, bufs[0])
    fire(0, bufs[0])

    zero16 = jnp.zeros((16,), jnp.float32)

    def _zs(r, _):
        for j in range(HID // 16):
            rows1[r, pl.ds(j * 16, 16)] = zero16
        as1[r, :] = zero16
        return _

    lax.fori_loop(0, ZR, _zs, None)
    r0 = s * (8 * ZR)
    for kk in range(NACC // NS // ZR):
        r = r0 + kk * ZR
        pltpu.sync_copy(rows1.at[pl.ds(0, ZR)], accS.at[pl.ds(r, ZR)])
        pltpu.sync_copy(as1.at[pl.ds(0, ZR)], accD.at[pl.ds(r, ZR)])
    plsc.subcore_barrier()

    def _pair(p, _):
        waiti(2 * p + 1, bufs[1])
        fire(2 * p + 1, bufs[1])
        waitg(bufs[0])
        compute(bufs[0])

        @pl.when(p < CPW // 2 - 1)
        def _():
            pfxi(2 * p + 2, bufs[0])

        waitg(bufs[1])
        compute(bufs[1])
        waitsc(bufs[0])

        @pl.when(p < CPW // 2 - 1)
        def _():
            waiti(2 * p + 2, bufs[0])
            fire(2 * p + 2, bufs[0])

        waitsc(bufs[1])

        @pl.when(p < CPW // 2 - 1)
        def _():
            pfxi(2 * p + 3, bufs[1])

        return _

    lax.fori_loop(0, CPW // 2, _pair, None)
    plsc.subcore_barrier()

    for kk in range(NACC // NS // ZR):
        r = r0 + kk * ZR
        pltpu.sync_copy(accS.at[pl.ds(r, ZR)], rows0.at[pl.ds(0, ZR)])
        pltpu.sync_copy(rows0.at[pl.ds(0, ZR)], outS.at[c, pl.ds(r, ZR)])
        pltpu.sync_copy(accD.at[pl.ds(r, ZR)], as0.at[pl.ds(0, ZR)])
        pltpu.sync_copy(as0.at[pl.ds(0, ZR)], outD.at[c, pl.ds(r, ZR)])


NCH2 = E2 // K


def _sc_edge(xs, ab16, sd):
    mesh = plsc.VectorSubcoreMesh(core_axis_name="c", subcore_axis_name="s",
                                  num_cores=NC, num_subcores=NS)
    buf = [
        pltpu.VMEM((2, K), jnp.int32),
        pltpu.VMEM((K, HID), jnp.float32),
        pltpu.VMEM((K, C), jnp.float32),
        pltpu.VMEM((K, C), jnp.float32),
        pltpu.VMEM((K,), jnp.int32),
        pltpu.SemaphoreType.DMA,
        pltpu.SemaphoreType.DMA,
        pltpu.SemaphoreType.DMA,
    ]
    return pl.kernel(
        _sc_edge_body,
        compiler_params=pltpu.CompilerParams(use_tc_tiling_on_sc=False),
        out_type=[
            jax.ShapeDtypeStruct((NC, NACC, HID), jnp.float32),
            jax.ShapeDtypeStruct((NC, NACC, C), jnp.float32),
        ],
        mesh=mesh,
        scratch_types=[
            pltpu.VMEM_SHARED((NACC, HID), jnp.float32),
            pltpu.VMEM_SHARED((NACC, C), jnp.float32),
            *buf, *buf,
        ],
    )(xs, ab16, sd)


def _postpre_body(accS_ref, accD_ref, xl_ref, cb_ref, exp_ref,
                  lW_ref, lb_ref, Ws_ref, Wd_ref, bds_ref, bdd_ref,
                  xl2_ref, xs_ref, ab_ref):
    S = accS_ref[0] + accS_ref[1]
    den = accD_ref[0] + accD_ref[1]
    d128 = jnp.dot(den, exp_ref[...], preferred_element_type=jnp.float32) + 1e-16
    hn = xl_ref[...] + S / d128 + cb_ref[...]
    hb = jnp.where(hn > 0, hn, jnp.exp(hn) - 1.0)
    xl2_ref[...] = jnp.dot(hb, lW_ref[...], preferred_element_type=jnp.float32) + lb_ref[...]
    xs = jnp.dot(hb, Ws_ref[...], preferred_element_type=jnp.float32)
    xs_ref[...] = xs
    xd = jnp.dot(hb, Wd_ref[...], preferred_element_type=jnp.float32)
    ab_ref[...] = (jnp.dot(xs, bds_ref[...], preferred_element_type=jnp.float32)
                   + jnp.dot(xd, bdd_ref[...], preferred_element_type=jnp.float32))


def _postpre_layer(accS, accD, xl, cb, expand, lW, lb, Ws, Wd, bds, bdd):
    grid = (N // RB,)
    full = lambda shape: pl.BlockSpec(shape, lambda i: tuple(0 for _ in shape))
    return pl.pallas_call(
        _postpre_body,
        grid=grid,
        in_specs=[
            pl.BlockSpec((NC, RB, HID), lambda i: (0, i, 0)),
            pl.BlockSpec((NC, RB, C), lambda i: (0, i, 0)),
            pl.BlockSpec((RB, HID), lambda i: (i, 0)),
            full((1, HID)), full((C, HID)),
            full((HID, HID)), full((1, HID)), full((HID, HID)), full((HID, HID)),
            full((HID, C)), full((HID, C)),
        ],
        out_specs=[
            pl.BlockSpec((RB, HID), lambda i: (i, 0)),
            pl.BlockSpec((RB, HID), lambda i: (i, 0)),
            pl.BlockSpec((RB, C), lambda i: (i, 0)),
        ],
        out_shape=[
            jax.ShapeDtypeStruct((N, HID), jnp.float32),
            jax.ShapeDtypeStruct((N, HID), jnp.float32),
            jax.ShapeDtypeStruct((NACC, C), jnp.float32),
        ],
    )(accS, accD, xl, cb, expand, lW, lb, Ws, Wd, bds, bdd)


def _final_body(accS_ref, accD_ref, xl_ref, cb_ref, exp_ref, fW_ref, fb_ref, o_ref):
    S = accS_ref[0] + accS_ref[1]
    den = accD_ref[0] + accD_ref[1]
    d128 = jnp.dot(den, exp_ref[...], preferred_element_type=jnp.float32) + 1e-16
    hn = xl_ref[...] + S / d128 + cb_ref[...]
    hn = jnp.where(hn > 0, hn, jnp.exp(hn) - 1.0)
    o_ref[...] = jnp.dot(hn, fW_ref[...], preferred_element_type=jnp.float32) + fb_ref[...]


def _post_layer(accS, accD, xl, cb, expand, fW, fb):
    grid = (N // RB,)
    full = lambda shape: pl.BlockSpec(shape, lambda i: tuple(0 for _ in shape))
    in_specs = [
        pl.BlockSpec((NC, RB, HID), lambda i: (0, i, 0)),
        pl.BlockSpec((NC, RB, C), lambda i: (0, i, 0)),
        pl.BlockSpec((RB, HID), lambda i: (i, 0)),
        full((1, HID)), full((C, HID)),
    ]
    args = [accS, accD, xl, cb, expand, fW, fb]
    body, out_d = _final_body, OUT
    in_specs += [full((HID, OUT)), full((1, OUT))]
    return pl.pallas_call(
        body,
        grid=grid,
        in_specs=in_specs,
        out_specs=pl.BlockSpec((RB, out_d), lambda i: (i, 0)),
        out_shape=jax.ShapeDtypeStruct((N, out_d), jnp.float32),
    )(*args)


def _blockdiag(att, off):
    bd = jnp.zeros((HID, C), jnp.float32)
    rows = jnp.arange(HID)
    return bd.at[rows, off + rows // C].set(att.reshape(HID))


_EXPAND_NP = np.zeros((C, HID), np.float32)
_EXPAND_NP[np.arange(HID) // C, np.arange(HID)] = 1.0


def kernel(x, edge_index, lW0, lb0, Ws0, Wd0, atts0, attd0, cb0,
           lW1, lb1, Ws1, Wd1, atts1, attd1, cb1, fW, fb):
    srcp = jnp.concatenate([edge_index[0], jnp.zeros((E2 - E,), jnp.int32)])
    dstp = jnp.concatenate([edge_index[1], jnp.full((E2 - E,), N, jnp.int32)])
    sd = jnp.stack([srcp.reshape(NCH2, K), dstp.reshape(NCH2, K)], axis=1)
    expand = jnp.asarray(_EXPAND_NP)

    xl0, xs0, ab0 = _pre_layer(
        x, lW0, lb0.reshape(1, HID), Ws0, Wd0,
        _blockdiag(atts0, 0), _blockdiag(attd0, 8))
    accS0, accD0 = _sc_edge(xs0, ab0, sd)
    xl1, xs1, ab1 = _postpre_layer(
        accS0, accD0, xl0, cb0.reshape(1, HID), expand,
        lW1, lb1.reshape(1, HID), Ws1, Wd1,
        _blockdiag(atts1, 0), _blockdiag(attd1, 8))
    accS1, accD1 = _sc_edge(xs1, ab1, sd)
    return _post_layer(accS1, accD1, xl1, cb1.reshape(1, HID), expand,
                       fW, fb.reshape(1, OUT))


# speedup vs baseline: 1.1247x; 1.0005x over previous
import functools
import numpy as np
import jax
import jax.numpy as jnp
from jax import lax
from jax.experimental import pallas as pl
from jax.experimental.pallas import tpu as pltpu
from jax.experimental.pallas import tpu_sc as plsc

N = 10000
E = 320000
D = 128
HID = 128
H = 8
C = HID // H
OUT = 128

NC, NS = 2, 16
NW = NC * NS
K = 96
CPW = 106
E2 = K * NW * CPW
NACC = 10240
ZR = 80

RB = 1000


def _pre_body(h_ref, lW_ref, lb_ref, Ws_ref, Wd_ref, bds_ref, bdd_ref,
              xl_ref, xs_ref, ab_ref):
    hb = h_ref[...]
    xl_ref[...] = jnp.dot(hb, lW_ref[...], preferred_element_type=jnp.float32) + lb_ref[...]
    xs = jnp.dot(hb, Ws_ref[...], preferred_element_type=jnp.float32)
    xs_ref[...] = xs
    xd = jnp.dot(hb, Wd_ref[...], preferred_element_type=jnp.float32)
    ab_ref[...] = (jnp.dot(xs, bds_ref[...], preferred_element_type=jnp.float32)
                   + jnp.dot(xd, bdd_ref[...], preferred_element_type=jnp.float32))


def _pre_layer(h, lW, lb, Ws, Wd, bds, bdd):
    grid = (N // RB,)
    full = lambda shape: pl.BlockSpec(shape, lambda i: (0, 0))
    return pl.pallas_call(
        _pre_body,
        grid=grid,
        in_specs=[
            pl.BlockSpec((RB, D), lambda i: (i, 0)),
            full((D, HID)), full((1, HID)), full((D, HID)), full((D, HID)),
            full((HID, C)), full((HID, C)),
        ],
        out_specs=[
            pl.BlockSpec((RB, HID), lambda i: (i, 0)),
            pl.BlockSpec((RB, HID), lambda i: (i, 0)),
            pl.BlockSpec((RB, C), lambda i: (i, 0)),
        ],
        out_shape=[
            jax.ShapeDtypeStruct((N, HID), jnp.float32),
            jax.ShapeDtypeStruct((N, HID), jnp.float32),
            jax.ShapeDtypeStruct((NACC, C), jnp.float32),
        ],
    )(h, lW, lb, Ws, Wd, bds, bdd)


def _sc_edge_body(xs_hbm, ab_hbm, sd_hbm,
                  outS, outD,
                  accS, accD,
                  sd0, rows0, as0, ad0, scd0, gsem0, ssem0, isem0,
                  sd1, rows1, as1, ad1, scd1, gsem1, ssem1, isem1):
    c = lax.axis_index("c")
    s = lax.axis_index("s")
    w = c * NS + s
    bufs = ((sd0, rows0, as0, ad0, scd0, gsem0, ssem0, isem0),
            (sd1, rows1, as1, ad1, scd1, gsem1, ssem1, isem1))

    def pfxi(j, b):
        sd_v, isem = b[0], b[7]
        pltpu.async_copy(sd_hbm.at[j * NW + w], sd_v, isem)

    def waiti(j, b):
        sd_v, isem = b[0], b[7]
        pltpu.make_async_copy(sd_hbm.at[j * NW + w], sd_v, isem).wait()

    def fire(j, b):
        sd_v, rows_v, asv, adv, gsem = b[0], b[1], b[2], b[3], b[5]
        pltpu.async_copy(xs_hbm.at[sd_v.at[0]], rows_v, gsem)
        pltpu.async_copy(ab_hbm.at[sd_v.at[0]], asv, gsem)
        pltpu.async_copy(ab_hbm.at[sd_v.at[1]], adv, gsem)

    def waitg(b):
        sd_v, rows_v, asv, adv, gsem = b[0], b[1], b[2], b[3], b[5]
        pltpu.make_async_copy(xs_hbm.at[sd_v.at[0]], rows_v, gsem).wait()
        pltpu.make_async_copy(ab_hbm.at[sd_v.at[0]], asv, gsem).wait()
        pltpu.make_async_copy(ab_hbm.at[sd_v.at[1]], adv, gsem).wait()

    perm = ((lax.iota(jnp.int32, 16) + 8) % 16).reshape(16, 1)
    gdn = lax.GatherDimensionNumbers(
        offset_dims=(), collapsed_slice_dims=(0,), start_index_map=(0,))

    def compute(b):
        sd_v, rows_v, asv, adv, scd, gsem, ssem = b[:7]

        @plsc.parallel_loop(0, K // 16, unroll=2)
        def _cpy(jj):
            scd[0, pl.ds(jj * 16, 16)] = sd_v[1, pl.ds(jj * 16, 16)]

        @plsc.parallel_loop(0, K, unroll=4)
        def _edge(e):
            adn = lax.gather(adv[e, :], perm, gdn, (1,),
                             mode=lax.GatherScatterMode.PROMISE_IN_BOUNDS)
            a = asv[e, :] + adn
            a = jnp.where(a >= 0, a, 0.2 * a)
            ex = jnp.exp(a)
            adv[e, :] = ex
            for h in range(H):
                rv = rows_v[e, pl.ds(h * 16, 16)]
                rows_v[e, pl.ds(h * 16, 16)] = rv * ex[h]

        pltpu.async_copy(rows_v, accS.at[scd.at[0]], ssem, add=True)
        pltpu.async_copy(adv, accD.at[scd.at[0]], ssem, add=True)

    def waitsc(b):
        sd_v, rows_v, asv, adv, scd, gsem, ssem = b[:7]
        pltpu.make_async_copy(rows_v, accS.at[scd.at[0]], ssem).wait()
        pltpu.make_async_copy(adv, accD.at[scd.at[0]], ssem).wait()

    pfxi(0, bufs[0])
    pfxi(1, bufs[1])
    waiti(0, bufs[0])
    fire(0, bufs[0])

    zero16 = jnp.zeros((16,), jnp.float32)

    def _zs(r, _):
        for j in range(HID // 16):
            rows1[r, pl.ds(j * 16, 16)] = zero16
        as1[r, :] = zero16
        return _

    lax.fori_loop(0, ZR, _zs, None)
    r0 = s * (8 * ZR)
    for kk in range(NACC // NS // ZR):
        r = r0 + kk * ZR
        pltpu.sync_copy(rows1.at[pl.ds(0, ZR)], accS.at[pl.ds(r, ZR)])
        pltpu.sync_copy(as1.at[pl.ds(0, ZR)], accD.at[pl.ds(r, ZR)])
    plsc.subcore_barrier()

    def _pair(p, _):
        waiti(2 * p + 1, bufs[1])
        fire(2 * p + 1, bufs[1])
        waitg(bufs[0])
        compute(bufs[0])

        @pl.when(p < CPW // 2 - 1)
        def _():
            pfxi(2 * p + 2, bufs[0])

        waitg(bufs[1])
        compute(bufs[1])
        waitsc(bufs[0])

        @pl.when(p < CPW // 2 - 1)
        def _():
            waiti(2 * p + 2, bufs[0])
            fire(2 * p + 2, bufs[0])

        waitsc(bufs[1])

        @pl.when(p < CPW // 2 - 1)
        def _():
            pfxi(2 * p + 3, bufs[1])

        return _

    lax.fori_loop(0, CPW // 2, _pair, None)
    plsc.subcore_barrier()

    for kk in range(NACC // NS // ZR):
        r = r0 + kk * ZR
        pltpu.sync_copy(accS.at[pl.ds(r, ZR)], rows0.at[pl.ds(0, ZR)])
        pltpu.sync_copy(rows0.at[pl.ds(0, ZR)], outS.at[c, pl.ds(r, ZR)])
        pltpu.sync_copy(accD.at[pl.ds(r, ZR)], as0.at[pl.ds(0, ZR)])
        pltpu.sync_copy(as0.at[pl.ds(0, ZR)], outD.at[c, pl.ds(r, ZR)])


NCH2 = E2 // K


def _sc_edge(xs, ab16, sd):
    mesh = plsc.VectorSubcoreMesh(core_axis_name="c", subcore_axis_name="s",
                                  num_cores=NC, num_subcores=NS)
    buf = [
        pltpu.VMEM((2, K), jnp.int32),
        pltpu.VMEM((K, HID), jnp.float32),
        pltpu.VMEM((K, C), jnp.float32),
        pltpu.VMEM((K, C), jnp.float32),
        pltpu.VMEM((1, K), jnp.int32),
        pltpu.SemaphoreType.DMA,
        pltpu.SemaphoreType.DMA,
        pltpu.SemaphoreType.DMA,
    ]
    return pl.kernel(
        _sc_edge_body,
        compiler_params=pltpu.CompilerParams(use_tc_tiling_on_sc=False),
        out_type=[
            jax.ShapeDtypeStruct((NC, NACC, HID), jnp.float32),
            jax.ShapeDtypeStruct((NC, NACC, C), jnp.float32),
        ],
        mesh=mesh,
        scratch_types=[
            pltpu.VMEM_SHARED((NACC, HID), jnp.float32),
            pltpu.VMEM_SHARED((NACC, C), jnp.float32),
            *buf, *buf,
        ],
    )(xs, ab16, sd)


def _postpre_body(accS_ref, accD_ref, xl_ref, cb_ref, exp_ref,
                  lW_ref, lb_ref, Ws_ref, Wd_ref, bds_ref, bdd_ref,
                  xl2_ref, xs_ref, ab_ref):
    S = accS_ref[0] + accS_ref[1]
    den = accD_ref[0] + accD_ref[1]
    d128 = jnp.dot(den, exp_ref[...], preferred_element_type=jnp.float32) + 1e-16
    hn = xl_ref[...] + S / d128 + cb_ref[...]
    hb = jnp.where(hn > 0, hn, jnp.exp(hn) - 1.0)
    xl2_ref[...] = jnp.dot(hb, lW_ref[...], preferred_element_type=jnp.float32) + lb_ref[...]
    xs = jnp.dot(hb, Ws_ref[...], preferred_element_type=jnp.float32)
    xs_ref[...] = xs
    xd = jnp.dot(hb, Wd_ref[...], preferred_element_type=jnp.float32)
    ab_ref[...] = (jnp.dot(xs, bds_ref[...], preferred_element_type=jnp.float32)
                   + jnp.dot(xd, bdd_ref[...], preferred_element_type=jnp.float32))


def _postpre_layer(accS, accD, xl, cb, expand, lW, lb, Ws, Wd, bds, bdd):
    grid = (N // RB,)
    full = lambda shape: pl.BlockSpec(shape, lambda i: tuple(0 for _ in shape))
    return pl.pallas_call(
        _postpre_body,
        grid=grid,
        in_specs=[
            pl.BlockSpec((NC, RB, HID), lambda i: (0, i, 0)),
            pl.BlockSpec((NC, RB, C), lambda i: (0, i, 0)),
            pl.BlockSpec((RB, HID), lambda i: (i, 0)),
            full((1, HID)), full((C, HID)),
            full((HID, HID)), full((1, HID)), full((HID, HID)), full((HID, HID)),
            full((HID, C)), full((HID, C)),
        ],
        out_specs=[
            pl.BlockSpec((RB, HID), lambda i: (i, 0)),
            pl.BlockSpec((RB, HID), lambda i: (i, 0)),
            pl.BlockSpec((RB, C), lambda i: (i, 0)),
        ],
        out_shape=[
            jax.ShapeDtypeStruct((N, HID), jnp.float32),
            jax.ShapeDtypeStruct((N, HID), jnp.float32),
            jax.ShapeDtypeStruct((NACC, C), jnp.float32),
        ],
    )(accS, accD, xl, cb, expand, lW, lb, Ws, Wd, bds, bdd)


def _final_body(accS_ref, accD_ref, xl_ref, cb_ref, exp_ref, fW_ref, fb_ref, o_ref):
    S = accS_ref[0] + accS_ref[1]
    den = accD_ref[0] + accD_ref[1]
    d128 = jnp.dot(den, exp_ref[...], preferred_element_type=jnp.float32) + 1e-16
    hn = xl_ref[...] + S / d128 + cb_ref[...]
    hn = jnp.where(hn > 0, hn, jnp.exp(hn) - 1.0)
    o_ref[...] = jnp.dot(hn, fW_ref[...], preferred_element_type=jnp.float32) + fb_ref[...]


def _post_layer(accS, accD, xl, cb, expand, fW, fb):
    grid = (N // RB,)
    full = lambda shape: pl.BlockSpec(shape, lambda i: tuple(0 for _ in shape))
    in_specs = [
        pl.BlockSpec((NC, RB, HID), lambda i: (0, i, 0)),
        pl.BlockSpec((NC, RB, C), lambda i: (0, i, 0)),
        pl.BlockSpec((RB, HID), lambda i: (i, 0)),
        full((1, HID)), full((C, HID)),
    ]
    args = [accS, accD, xl, cb, expand, fW, fb]
    body, out_d = _final_body, OUT
    in_specs += [full((HID, OUT)), full((1, OUT))]
    return pl.pallas_call(
        body,
        grid=grid,
        in_specs=in_specs,
        out_specs=pl.BlockSpec((RB, out_d), lambda i: (i, 0)),
        out_shape=jax.ShapeDtypeStruct((N, out_d), jnp.float32),
    )(*args)


def _blockdiag(att, off):
    bd = jnp.zeros((HID, C), jnp.float32)
    rows = jnp.arange(HID)
    return bd.at[rows, off + rows // C].set(att.reshape(HID))


_EXPAND_NP = np.zeros((C, HID), np.float32)
_EXPAND_NP[np.arange(HID) // C, np.arange(HID)] = 1.0


def kernel(x, edge_index, lW0, lb0, Ws0, Wd0, atts0, attd0, cb0,
           lW1, lb1, Ws1, Wd1, atts1, attd1, cb1, fW, fb):
    srcp = jnp.concatenate([edge_index[0], jnp.zeros((E2 - E,), jnp.int32)])
    dstp = jnp.concatenate([edge_index[1], jnp.full((E2 - E,), N, jnp.int32)])
    sd = jnp.stack([srcp.reshape(NCH2, K), dstp.reshape(NCH2, K)], axis=1)
    expand = jnp.asarray(_EXPAND_NP)

    xl0, xs0, ab0 = _pre_layer(
        x, lW0, lb0.reshape(1, HID), Ws0, Wd0,
        _blockdiag(atts0, 0), _blockdiag(attd0, 8))
    accS0, accD0 = _sc_edge(xs0, ab0, sd)
    xl1, xs1, ab1 = _postpre_layer(
        accS0, accD0, xl0, cb0.reshape(1, HID), expand,
        lW1, lb1.reshape(1, HID), Ws1, Wd1,
        _blockdiag(atts1, 0), _blockdiag(attd1, 8))
    accS1, accD1 = _sc_edge(xs1, ab1, sd)
    return _post_layer(accS1, accD1, xl1, cb1.reshape(1, HID), expand,
                       fW, fb.reshape(1, OUT))


# speedup vs baseline: 1.5246x; 1.3555x over previous
import functools
import numpy as np
import jax
import jax.numpy as jnp
from jax import lax
from jax.experimental import pallas as pl
from jax.experimental.pallas import tpu as pltpu
from jax.experimental.pallas import tpu_sc as plsc

N = 10000
E = 320000
D = 128
HID = 128
H = 8
C = HID // H
OUT = 128

NC, NS = 2, 16
NW = NC * NS
K = 112
CPW = 90
E2 = K * NW * CPW
NACC = 10240
ZR = 80

RB = 1000


def _pre_body(h_ref, lW_ref, lb_ref, Ws_ref, Wd_ref, bds_ref, bdd_ref,
              xl_ref, xs_ref, ab_ref):
    hb = h_ref[...]
    xl_ref[...] = jnp.dot(hb, lW_ref[...], preferred_element_type=jnp.float32) + lb_ref[...]
    xs = jnp.dot(hb, Ws_ref[...], preferred_element_type=jnp.float32)
    xs_ref[...] = xs
    xd = jnp.dot(hb, Wd_ref[...], preferred_element_type=jnp.float32)
    ab_ref[...] = (jnp.dot(xs, bds_ref[...], preferred_element_type=jnp.float32)
                   + jnp.dot(xd, bdd_ref[...], preferred_element_type=jnp.float32))


def _pre_layer(h, lW, lb, Ws, Wd, bds, bdd):
    grid = (N // RB,)
    full = lambda shape: pl.BlockSpec(shape, lambda i: (0, 0))
    return pl.pallas_call(
        _pre_body,
        grid=grid,
        in_specs=[
            pl.BlockSpec((RB, D), lambda i: (i, 0)),
            full((D, HID)), full((1, HID)), full((D, HID)), full((D, HID)),
            full((HID, C)), full((HID, C)),
        ],
        out_specs=[
            pl.BlockSpec((RB, HID), lambda i: (i, 0)),
            pl.BlockSpec((RB, HID), lambda i: (i, 0)),
            pl.BlockSpec((RB, C), lambda i: (i, 0)),
        ],
        out_shape=[
            jax.ShapeDtypeStruct((N, HID), jnp.float32),
            jax.ShapeDtypeStruct((N, HID), jnp.float32),
            jax.ShapeDtypeStruct((NACC, C), jnp.float32),
        ],
    )(h, lW, lb, Ws, Wd, bds, bdd)


def _sc_edge_body(xs_hbm, ab_hbm, sd_hbm,
                  outS, outD,
                  accS, accD,
                  sd0, rows0, as0, ad0, scd0, gsem0, ssem0, isem0,
                  sd1, rows1, as1, ad1, scd1, gsem1, ssem1, isem1):
    c = lax.axis_index("c")
    s = lax.axis_index("s")
    w = c * NS + s
    bufs = ((sd0, rows0, as0, ad0, scd0, gsem0, ssem0, isem0),
            (sd1, rows1, as1, ad1, scd1, gsem1, ssem1, isem1))

    def pfxi(j, b):
        sd_v, isem = b[0], b[7]
        pltpu.async_copy(sd_hbm.at[j * NW + w], sd_v, isem)

    def waiti(j, b):
        sd_v, isem = b[0], b[7]
        pltpu.make_async_copy(sd_hbm.at[j * NW + w], sd_v, isem).wait()

    def fire(j, b):
        sd_v, rows_v, asv, adv, gsem = b[0], b[1], b[2], b[3], b[5]
        pltpu.async_copy(xs_hbm.at[sd_v.at[0]], rows_v, gsem)
        pltpu.async_copy(ab_hbm.at[sd_v.at[0]], asv, gsem)
        pltpu.async_copy(ab_hbm.at[sd_v.at[1]], adv, gsem)

    def waitg(b):
        sd_v, rows_v, asv, adv, gsem = b[0], b[1], b[2], b[3], b[5]
        pltpu.make_async_copy(xs_hbm.at[sd_v.at[0]], rows_v, gsem).wait()
        pltpu.make_async_copy(ab_hbm.at[sd_v.at[0]], asv, gsem).wait()
        pltpu.make_async_copy(ab_hbm.at[sd_v.at[1]], adv, gsem).wait()

    perm = ((lax.iota(jnp.int32, 16) + 8) % 16).reshape(16, 1)
    gdn = lax.GatherDimensionNumbers(
        offset_dims=(), collapsed_slice_dims=(0,), start_index_map=(0,))

    def compute(b):
        sd_v, rows_v, asv, adv, scd, gsem, ssem = b[:7]

        @plsc.parallel_loop(0, K // 16, unroll=2)
        def _cpy(jj):
            scd[0, pl.ds(jj * 16, 16)] = sd_v[1, pl.ds(jj * 16, 16)]

        @plsc.parallel_loop(0, K, unroll=4)
        def _edge(e):
            adn = lax.gather(adv[e, :], perm, gdn, (1,),
                             mode=lax.GatherScatterMode.PROMISE_IN_BOUNDS)
            a = asv[e, :] + adn
            a = jnp.where(a >= 0, a, 0.2 * a)
            ex = jnp.exp(a)
            adv[e, :] = ex
            for h in range(H):
                rv = rows_v[e, pl.ds(h * 16, 16)]
                rows_v[e, pl.ds(h * 16, 16)] = rv * ex[h]

        pltpu.async_copy(rows_v, accS.at[scd.at[0]], ssem, add=True)
        pltpu.async_copy(adv, accD.at[scd.at[0]], ssem, add=True)

    def waitsc(b):
        sd_v, rows_v, asv, adv, scd, gsem, ssem = b[:7]
        pltpu.make_async_copy(rows_v, accS.at[scd.at[0]], ssem).wait()
        pltpu.make_async_copy(adv, accD.at[scd.at[0]], ssem).wait()

    pfxi(0, bufs[0])
    pfxi(1, bufs[1])
    waiti(0, bufs[0])
    fire(0, bufs[0])

    zero16 = jnp.zeros((16,), jnp.float32)

    def _zs(r, _):
        for j in range(HID // 16):
            rows1[r, pl.ds(j * 16, 16)] = zero16
        as1[r, :] = zero16
        return _

    lax.fori_loop(0, ZR, _zs, None)
    r0 = s * (8 * ZR)
    for kk in range(NACC // NS // ZR):
        r = r0 + kk * ZR
        pltpu.sync_copy(rows1.at[pl.ds(0, ZR)], accS.at[pl.ds(r, ZR)])
        pltpu.sync_copy(as1.at[pl.ds(0, ZR)], accD.at[pl.ds(r, ZR)])
    plsc.subcore_barrier()

    def _pair(p, _):
        waiti(2 * p + 1, bufs[1])
        fire(2 * p + 1, bufs[1])
        waitg(bufs[0])
        compute(bufs[0])

        @pl.when(p < CPW // 2 - 1)
        def _():
            pfxi(2 * p + 2, bufs[0])

        waitg(bufs[1])
        compute(bufs[1])
        waitsc(bufs[0])

        @pl.when(p < CPW // 2 - 1)
        def _():
            waiti(2 * p + 2, bufs[0])
            fire(2 * p + 2, bufs[0])

        waitsc(bufs[1])

        @pl.when(p < CPW // 2 - 1)
        def _():
            pfxi(2 * p + 3, bufs[1])

        return _

    lax.fori_loop(0, CPW // 2, _pair, None)
    plsc.subcore_barrier()

    for kk in range(NACC // NS // ZR):
        r = r0 + kk * ZR
        pltpu.sync_copy(accS.at[pl.ds(r, ZR)], rows0.at[pl.ds(0, ZR)])
        pltpu.sync_copy(rows0.at[pl.ds(0, ZR)], outS.at[c, pl.ds(r, ZR)])
        pltpu.sync_copy(accD.at[pl.ds(r, ZR)], as0.at[pl.ds(0, ZR)])
        pltpu.sync_copy(as0.at[pl.ds(0, ZR)], outD.at[c, pl.ds(r, ZR)])


NCH2 = E2 // K


def _sc_edge(xs, ab16, sd):
    mesh = plsc.VectorSubcoreMesh(core_axis_name="c", subcore_axis_name="s",
                                  num_cores=NC, num_subcores=NS)
    buf = [
        pltpu.VMEM((2, K), jnp.int32),
        pltpu.VMEM((K, HID), jnp.float32),
        pltpu.VMEM((K, C), jnp.float32),
        pltpu.VMEM((K, C), jnp.float32),
        pltpu.VMEM((1, K), jnp.int32),
        pltpu.SemaphoreType.DMA,
        pltpu.SemaphoreType.DMA,
        pltpu.SemaphoreType.DMA,
    ]
    return pl.kernel(
        _sc_edge_body,
        compiler_params=pltpu.CompilerParams(use_tc_tiling_on_sc=False),
        out_type=[
            jax.ShapeDtypeStruct((NC, NACC, HID), jnp.float32),
            jax.ShapeDtypeStruct((NC, NACC, C), jnp.float32),
        ],
        mesh=mesh,
        scratch_types=[
            pltpu.VMEM_SHARED((NACC, HID), jnp.float32),
            pltpu.VMEM_SHARED((NACC, C), jnp.float32),
            *buf, *buf,
        ],
    )(xs, ab16, sd)


def _postpre_body(accS_ref, accD_ref, xl_ref, cb_ref, exp_ref,
                  lW_ref, lb_ref, Ws_ref, Wd_ref, bds_ref, bdd_ref,
                  xl2_ref, xs_ref, ab_ref):
    S = accS_ref[0] + accS_ref[1]
    den = accD_ref[0] + accD_ref[1]
    d128 = jnp.dot(den, exp_ref[...], preferred_element_type=jnp.float32) + 1e-16
    hn = xl_ref[...] + S / d128 + cb_ref[...]
    hb = jnp.where(hn > 0, hn, jnp.exp(hn) - 1.0)
    xl2_ref[...] = jnp.dot(hb, lW_ref[...], preferred_element_type=jnp.float32) + lb_ref[...]
    xs = jnp.dot(hb, Ws_ref[...], preferred_element_type=jnp.float32)
    xs_ref[...] = xs
    xd = jnp.dot(hb, Wd_ref[...], preferred_element_type=jnp.float32)
    ab_ref[...] = (jnp.dot(xs, bds_ref[...], preferred_element_type=jnp.float32)
                   + jnp.dot(xd, bdd_ref[...], preferred_element_type=jnp.float32))


def _postpre_layer(accS, accD, xl, cb, expand, lW, lb, Ws, Wd, bds, bdd):
    grid = (N // RB,)
    full = lambda shape: pl.BlockSpec(shape, lambda i: tuple(0 for _ in shape))
    return pl.pallas_call(
        _postpre_body,
        grid=grid,
        in_specs=[
            pl.BlockSpec((NC, RB, HID), lambda i: (0, i, 0)),
            pl.BlockSpec((NC, RB, C), lambda i: (0, i, 0)),
            pl.BlockSpec((RB, HID), lambda i: (i, 0)),
            full((1, HID)), full((C, HID)),
            full((HID, HID)), full((1, HID)), full((HID, HID)), full((HID, HID)),
            full((HID, C)), full((HID, C)),
        ],
        out_specs=[
            pl.BlockSpec((RB, HID), lambda i: (i, 0)),
            pl.BlockSpec((RB, HID), lambda i: (i, 0)),
            pl.BlockSpec((RB, C), lambda i: (i, 0)),
        ],
        out_shape=[
            jax.ShapeDtypeStruct((N, HID), jnp.float32),
            jax.ShapeDtypeStruct((N, HID), jnp.float32),
            jax.ShapeDtypeStruct((NACC, C), jnp.float32),
        ],
    )(accS, accD, xl, cb, expand, lW, lb, Ws, Wd, bds, bdd)


def _final_body(accS_ref, accD_ref, xl_ref, cb_ref, exp_ref, fW_ref, fb_ref, o_ref):
    S = accS_ref[0] + accS_ref[1]
    den = accD_ref[0] + accD_ref[1]
    d128 = jnp.dot(den, exp_ref[...], preferred_element_type=jnp.float32) + 1e-16
    hn = xl_ref[...] + S / d128 + cb_ref[...]
    hn = jnp.where(hn > 0, hn, jnp.exp(hn) - 1.0)
    o_ref[...] = jnp.dot(hn, fW_ref[...], preferred_element_type=jnp.float32) + fb_ref[...]


def _post_layer(accS, accD, xl, cb, expand, fW, fb):
    grid = (N // RB,)
    full = lambda shape: pl.BlockSpec(shape, lambda i: tuple(0 for _ in shape))
    in_specs = [
        pl.BlockSpec((NC, RB, HID), lambda i: (0, i, 0)),
        pl.BlockSpec((NC, RB, C), lambda i: (0, i, 0)),
        pl.BlockSpec((RB, HID), lambda i: (i, 0)),
        full((1, HID)), full((C, HID)),
    ]
    args = [accS, accD, xl, cb, expand, fW, fb]
    body, out_d = _final_body, OUT
    in_specs += [full((HID, OUT)), full((1, OUT))]
    return pl.pallas_call(
        body,
        grid=grid,
        in_specs=in_specs,
        out_specs=pl.BlockSpec((RB, out_d), lambda i: (i, 0)),
        out_shape=jax.ShapeDtypeStruct((N, out_d), jnp.float32),
    )(*args)


def _blockdiag(att, off):
    bd = jnp.zeros((HID, C), jnp.float32)
    rows = jnp.arange(HID)
    return bd.at[rows, off + rows // C].set(att.reshape(HID))


_EXPAND_NP = np.zeros((C, HID), np.float32)
_EXPAND_NP[np.arange(HID) // C, np.arange(HID)] = 1.0


def kernel(x, edge_index, lW0, lb0, Ws0, Wd0, atts0, attd0, cb0,
           lW1, lb1, Ws1, Wd1, atts1, attd1, cb1, fW, fb):
    srcp = jnp.concatenate([edge_index[0], jnp.zeros((E2 - E,), jnp.int32)])
    dstp = jnp.concatenate([edge_index[1], jnp.full((E2 - E,), N, jnp.int32)])
    sd = jnp.stack([srcp.reshape(NCH2, K), dstp.reshape(NCH2, K)], axis=1)
    expand = jnp.asarray(_EXPAND_NP)

    xl0, xs0, ab0 = _pre_layer(
        x, lW0, lb0.reshape(1, HID), Ws0, Wd0,
        _blockdiag(atts0, 0), _blockdiag(attd0, 8))
    accS0, accD0 = _sc_edge(xs0, ab0, sd)
    xl1, xs1, ab1 = _postpre_layer(
        accS0, accD0, xl0, cb0.reshape(1, HID), expand,
        lW1, lb1.reshape(1, HID), Ws1, Wd1,
        _blockdiag(atts1, 0), _blockdiag(attd1, 8))
    accS1, accD1 = _sc_edge(xs1, ab1, sd)
    return _post_layer(accS1, accD1, xl1, cb1.reshape(1, HID), expand,
                       fW, fb.reshape(1, OUT))
